# baseline (speedup 1.0000x reference)
import functools

import jax
import jax.numpy as jnp
from jax import lax
from jax.experimental import pallas as pl
from jax.experimental.pallas import tpu as pltpu

_MiB = 1024 * 1024


def _wwt_in_kernel(win, wout):
    scale = win / wout
    o = lax.broadcasted_iota(jnp.int32, (win, wout), 1).astype(jnp.float32)
    k = lax.broadcasted_iota(jnp.int32, (win, wout), 0).astype(jnp.float32)
    src = jnp.maximum((o + 0.5) * scale - 0.5, 0.0)
    i0 = jnp.minimum(jnp.floor(src), float(win - 1))
    w1 = src - i0
    i1 = jnp.minimum(i0 + 1.0, float(win - 1))
    return (jnp.where(k == i0, 1.0 - w1, 0.0)
            + jnp.where(k == i1, w1, 0.0))


def _up_concat_kernel(x_ref, skip_ref, out_ref, xh_ref, *, nx_tiles):
    t = pl.program_id(1)

    @pl.when(t < nx_tiles)
    def _compute():
        ct, hin, win = x_ref.shape
        wout = 2 * win
        hout = 2 * hin
        x3 = x_ref[...]
        lo = x3[:, :-1, :]
        hi = x3[:, 1:, :]
        xh_ref[:, pl.Slice(2, hin - 1, 2), :] = 0.25 * lo + 0.75 * hi
        xh_ref[:, pl.Slice(1, hin - 1, 2), :] = 0.75 * lo + 0.25 * hi
        xh_ref[:, 0:1, :] = x3[:, 0:1, :]
        xh_ref[:, hout - 1:hout, :] = x3[:, hin - 1:hin, :]
        wwt = _wwt_in_kernel(win, wout)
        xh2d = xh_ref[...].reshape(ct * hout, win)
        out_ref[...] = jnp.dot(
            xh2d, wwt, preferred_element_type=jnp.float32,
        ).reshape(ct, hout, wout)

    @pl.when(t >= nx_tiles)
    def _copy_skip():
        out_ref[...] = skip_ref[...].astype(out_ref.dtype)


def kernel(x, skip):
    B, Cx, Hin, Win = x.shape
    Bs, Cs, Hout, Wout = skip.shape
    assert B == Bs and Hout == 2 * Hin and Wout == 2 * Win
    if skip.dtype != x.dtype:
        skip = skip.astype(x.dtype)

    bpe = jnp.dtype(x.dtype).itemsize
    ct = 128 if Cx % 128 == 0 else max(
        d for d in range(1, Cx + 1) if Cx % d == 0 and d <= 128)
    nx = Cx // ct
    ns = -(-Cs // ct)
    grid = (B, nx + ns)

    out_shape = jax.ShapeDtypeStruct((B, Cx + Cs, Hout, Wout), x.dtype)
    flops = int(2 * B * Cx * Hin * Win * Wout + 4 * B * Cx * Hout * Wout)
    bytes_accessed = int(x.size * bpe + skip.size * bpe
                         + B * (Cx + Cs) * Hout * Wout * bpe)
    cost = pl.CostEstimate(flops=flops, transcendentals=0,
                           bytes_accessed=bytes_accessed)
    cparams = pltpu.CompilerParams(
        dimension_semantics=("parallel", "parallel"),
        vmem_limit_bytes=60 * _MiB)

    grid_spec = pltpu.PrefetchScalarGridSpec(
        num_scalar_prefetch=0,
        grid=grid,
        scratch_shapes=[pltpu.VMEM((ct, Hout, Win), jnp.float32)],
        in_specs=[
            pl.BlockSpec((None, ct, Hin, Win),
                         lambda b, t: (b, jnp.minimum(t, nx - 1), 0, 0)),
            pl.BlockSpec((None, ct, Hout, Wout),
                         lambda b, t: (b, jnp.maximum(t - nx, 0), 0, 0)),
        ],
        out_specs=pl.BlockSpec((None, ct, Hout, Wout),
                               lambda b, t: (b, t, 0, 0)),
    )
    return pl.pallas_call(
        functools.partial(_up_concat_kernel, nx_tiles=nx),
        out_shape=out_shape,
        grid_spec=grid_spec,
        compiler_params=cparams,
        cost_estimate=cost,
    )(x, skip)
